# baseline (speedup 1.0000x reference)
import functools

import jax
import jax.numpy as jnp
from jax import lax
from jax.experimental import pallas as pl
from jax.experimental.pallas import tpu as pltpu
from jax.experimental.pallas import tpu_sc as plsc

_CH = 8
_G = 128


@functools.lru_cache(maxsize=None)
def _make_gather(bsz, seq, d):
    info = plsc.get_sparse_core_info()
    nc, ns = info.num_cores, info.num_subcores
    nw = nc * ns
    rows_per_w = bsz // nw
    chunks = rows_per_w // _CH
    assert rows_per_w * nw == bsz and chunks * _CH == rows_per_w
    segs = []
    off = 0
    while off < seq:
        segs.append((off, min(_G, seq - off)))
        off += min(_G, seq - off)
    mesh = plsc.VectorSubcoreMesh(core_axis_name="c", subcore_axis_name="s")

    @functools.partial(
        pl.kernel,
        mesh=mesh,
        out_type=jax.ShapeDtypeStruct((bsz, seq, d), jnp.float32),
        scratch_types=[
            pltpu.VMEM((_CH, seq), jnp.int32),
            pltpu.VMEM((_CH, seq, d), jnp.float32),
            pltpu.SemaphoreType.DMA,
        ],
        compiler_params=pltpu.CompilerParams(use_tc_tiling_on_sc=False),
    )
    def gather_kernel(idx_hbm, table_hbm, out_hbm, idx_v, rows_v, sem):
        wid = lax.axis_index("s") * nc + lax.axis_index("c")
        base = wid * rows_per_w

        def body(g, carry):
            row = base + g * _CH
            pltpu.sync_copy(idx_hbm.at[pl.ds(row, _CH)], idx_v)
            descs = []
            for j in range(_CH):
                for (o, n) in segs:
                    descs.append(pltpu.async_copy(
                        table_hbm.at[idx_v.at[j].at[pl.ds(o, n)]],
                        rows_v.at[j].at[pl.ds(o, n)],
                        sem))
            for dsc in descs:
                dsc.wait()
            pltpu.sync_copy(rows_v, out_hbm.at[pl.ds(row, _CH)])
            return carry

        lax.fori_loop(0, chunks, body, 0)

    return gather_kernel


def kernel(token_ids, weight):
    b, s = token_ids.shape
    d = weight.shape[1]
    ids = token_ids.astype(jnp.int32)
    return _make_gather(b, s, d)(ids, weight)
